# baseline (speedup 1.0000x reference)
import functools

import jax
import jax.numpy as jnp
from jax import lax
from jax.experimental import pallas as pl
from jax.experimental.pallas import tpu as pltpu
from jax.experimental.pallas import tpu_sc as plsc

Q = 1024
D = 128
K = 100000
NB = 20
THRESH = 0.8

KB = 1024
NT = 98
KP = NT * KB
BLK = 128
R = KP // BLK
NEG = -3.0e38
BIGI = 2**30

SC_NC = 2
SC_NS = 16
SC_NW = SC_NC * SC_NS
G_TOTAL = Q * NB
G_PER_W = G_TOTAL // SC_NW
G_CHUNK = 128
G_NCHUNK = G_PER_W // G_CHUNK


def _phase_a_body(q_ref, k_ref, sims_ref, bmax_ref):
    t = pl.program_id(0)
    s = lax.dot_general(
        q_ref[...], k_ref[...],
        (((1,), (1,)), ((), ())),
        preferred_element_type=jnp.float32,
    )
    col = t * KB + lax.broadcasted_iota(jnp.int32, (Q, KB), 1)
    s = jnp.where(col < K, s, NEG)
    sims_ref[...] = s
    for j in range(KB // BLK):
        bmax_ref[0, :, j : j + 1] = jnp.max(
            s[:, j * BLK : (j + 1) * BLK], axis=1, keepdims=True
        )


def _phase_b_body(m_ref, bids_ref, grow_ref):
    m = m_ref[...]
    bi = lax.broadcasted_iota(jnp.int32, (Q, R), 1)
    qi = lax.broadcasted_iota(jnp.int32, (Q, 1), 0)
    for j in range(NB):
        mx = jnp.max(m, axis=1, keepdims=True)
        bid = jnp.min(jnp.where(m == mx, bi, BIGI), axis=1, keepdims=True)
        bids_ref[:, j : j + 1] = bid
        grow_ref[:, j : j + 1] = qi * R + bid
        m = jnp.where(bi == bid, NEG, m)


def _phase_d_body(cand_ref, gidx_ref, vals_ref, idx_ref, msk_ref):
    v = cand_ref[...]
    g = gidx_ref[...]
    v = jnp.where(g < K, v, NEG)
    for j in range(NB):
        m = jnp.max(jnp.max(v, axis=2), axis=1, keepdims=True)
        sel = v == m[:, :, None]
        ci = jnp.min(jnp.min(jnp.where(sel, g, BIGI), axis=2), axis=1,
                     keepdims=True)
        vals_ref[:, j : j + 1] = m
        idx_ref[:, j : j + 1] = ci
        msk_ref[:, j : j + 1] = m >= THRESH
        v = jnp.where(g == ci[:, :, None], NEG, v)


def _sc_gather(sims_flat, grow3d):
    mesh = plsc.VectorSubcoreMesh(core_axis_name="c", subcore_axis_name="s")

    @functools.partial(
        pl.kernel,
        mesh=mesh,
        out_type=jax.ShapeDtypeStruct((G_TOTAL, BLK), jnp.float32),
        scratch_types=[
            pltpu.VMEM((G_NCHUNK, G_CHUNK), jnp.int32),
            pltpu.VMEM((G_PER_W, BLK), jnp.float32),
            pltpu.SemaphoreType.DMA,
        ],
    )
    def gather_kernel(table_hbm, idx_hbm, out_hbm, idx_v, rows_v, sem):
        wid = lax.axis_index("s") * SC_NC + lax.axis_index("c")
        pltpu.sync_copy(idx_hbm.at[wid], idx_v)
        for j in range(G_NCHUNK):
            pltpu.async_copy(
                table_hbm.at[idx_v.at[j]],
                rows_v.at[pl.ds(j * G_CHUNK, G_CHUNK)],
                sem,
            ).wait()
        pltpu.sync_copy(rows_v, out_hbm.at[pl.ds(wid * G_PER_W, G_PER_W)])

    return gather_kernel(sims_flat, grow3d)


def kernel(queries, keys):
    sims, bmax = pl.pallas_call(
        _phase_a_body,
        grid=(NT,),
        in_specs=[
            pl.BlockSpec((Q, D), lambda t: (0, 0)),
            pl.BlockSpec((KB, D), lambda t: (t, 0)),
        ],
        out_specs=[
            pl.BlockSpec((Q, KB), lambda t: (0, t)),
            pl.BlockSpec((1, Q, KB // BLK), lambda t: (t, 0, 0)),
        ],
        out_shape=[
            jax.ShapeDtypeStruct((Q, KP), jnp.float32),
            jax.ShapeDtypeStruct((NT, Q, KB // BLK), jnp.float32),
        ],
        compiler_params=pltpu.CompilerParams(
            dimension_semantics=("arbitrary",),
        ),
    )(queries, keys)

    bmax = bmax.transpose(1, 0, 2).reshape(Q, R)

    bids, grow = pl.pallas_call(
        _phase_b_body,
        out_shape=[
            jax.ShapeDtypeStruct((Q, NB), jnp.int32),
            jax.ShapeDtypeStruct((Q, NB), jnp.int32),
        ],
    )(bmax)

    cand = _sc_gather(
        sims.reshape(Q * R, BLK),
        grow.reshape(SC_NW, G_NCHUNK, G_CHUNK),
    )

    if True:
        return (cand[:Q, :NB], bids + grow, (bids >= 0))

    gidx = bids[:, :, None] * BLK + jnp.arange(BLK, dtype=jnp.int32)

    QT = 256
    vals, idx, msk = pl.pallas_call(
        _phase_d_body,
        grid=(Q // QT,),
        in_specs=[
            pl.BlockSpec((QT, NB, BLK), lambda i: (i, 0, 0)),
            pl.BlockSpec((QT, NB, BLK), lambda i: (i, 0, 0)),
        ],
        out_specs=[
            pl.BlockSpec((QT, NB), lambda i: (i, 0)),
            pl.BlockSpec((QT, NB), lambda i: (i, 0)),
            pl.BlockSpec((QT, NB), lambda i: (i, 0)),
        ],
        out_shape=[
            jax.ShapeDtypeStruct((Q, NB), jnp.float32),
            jax.ShapeDtypeStruct((Q, NB), jnp.int32),
            jax.ShapeDtypeStruct((Q, NB), jnp.bool_),
        ],
        compiler_params=pltpu.CompilerParams(
            dimension_semantics=("arbitrary",),
        ),
    )(cand.reshape(Q, NB, BLK), gidx)

    return vals, idx, msk


# speedup vs baseline: 1.4558x; 1.4558x over previous
import functools

import jax
import jax.numpy as jnp
from jax import lax
from jax.experimental import pallas as pl
from jax.experimental.pallas import tpu as pltpu
from jax.experimental.pallas import tpu_sc as plsc

Q = 1024
D = 128
K = 100000
NB = 20
THRESH = 0.8

KB = 1024
NT = 98
KP = NT * KB
BLK = 128
R = KP // BLK
NEG = -3.0e38
BIGI = 2**30

SC_NC = 2
SC_NS = 16
SC_NW = SC_NC * SC_NS
G_TOTAL = Q * NB
G_PER_W = G_TOTAL // SC_NW
G_CHUNK = 128
G_NCHUNK = G_PER_W // G_CHUNK


def _phase_a_body(q_ref, k_ref, sims_ref, bmax_ref):
    t = pl.program_id(0)
    s = lax.dot_general(
        q_ref[...], k_ref[...],
        (((1,), (1,)), ((), ())),
        preferred_element_type=jnp.float32,
    )
    col = t * KB + lax.broadcasted_iota(jnp.int32, (Q, KB), 1)
    s = jnp.where(col < K, s, NEG)
    for j in range(KB // BLK):
        blk = s[:, j * BLK : (j + 1) * BLK]
        sims_ref[j, :, :] = blk
        bmax_ref[0, :, j : j + 1] = jnp.max(blk, axis=1, keepdims=True)


def _phase_b_body(m_ref, bids_ref, grow_ref):
    m = m_ref[...]
    bi = lax.broadcasted_iota(jnp.int32, (Q, R), 1)
    qi = lax.broadcasted_iota(jnp.int32, (Q, 1), 0)
    for j in range(NB):
        mx = jnp.max(m, axis=1, keepdims=True)
        bid = jnp.min(jnp.where(m == mx, bi, BIGI), axis=1, keepdims=True)
        bids_ref[:, j : j + 1] = bid
        grow_ref[:, j : j + 1] = bid * Q + qi
        m = jnp.where(bi == bid, NEG, m)


def _phase_d_body(cand_ref, gidx_ref, vals_ref, idx_ref, msk_ref):
    v = cand_ref[...]
    g = gidx_ref[...]
    v = jnp.where(g < K, v, NEG)
    for j in range(NB):
        m = jnp.max(v, axis=1, keepdims=True)
        ci = jnp.min(jnp.where(v == m, g, BIGI), axis=1, keepdims=True)
        vals_ref[:, j : j + 1] = m
        idx_ref[:, j : j + 1] = ci
        msk_ref[:, j : j + 1] = m >= THRESH
        v = jnp.where(g == ci, NEG, v)


def _sc_gather(sims_flat, grow3d):
    mesh = plsc.VectorSubcoreMesh(core_axis_name="c", subcore_axis_name="s")

    @functools.partial(
        pl.kernel,
        mesh=mesh,
        out_type=jax.ShapeDtypeStruct((G_TOTAL, BLK), jnp.float32),
        scratch_types=[
            pltpu.VMEM((G_NCHUNK, G_CHUNK), jnp.int32),
            pltpu.VMEM((G_PER_W, BLK), jnp.float32),
            pltpu.SemaphoreType.DMA,
        ],
    )
    def gather_kernel(table_hbm, idx_hbm, out_hbm, idx_v, rows_v, sem):
        wid = lax.axis_index("s") * SC_NC + lax.axis_index("c")
        pltpu.sync_copy(idx_hbm.at[wid], idx_v)
        for j in range(G_NCHUNK):
            pltpu.async_copy(
                table_hbm.at[idx_v.at[j]],
                rows_v.at[pl.ds(j * G_CHUNK, G_CHUNK)],
                sem,
            ).wait()
        pltpu.sync_copy(rows_v, out_hbm.at[pl.ds(wid * G_PER_W, G_PER_W)])

    return gather_kernel(sims_flat, grow3d)


def kernel(queries, keys):
    sims, bmax = pl.pallas_call(
        _phase_a_body,
        grid=(NT,),
        in_specs=[
            pl.BlockSpec((Q, D), lambda t: (0, 0)),
            pl.BlockSpec((KB, D), lambda t: (t, 0)),
        ],
        out_specs=[
            pl.BlockSpec((KB // BLK, Q, BLK), lambda t: (t, 0, 0)),
            pl.BlockSpec((1, Q, KB // BLK), lambda t: (t, 0, 0)),
        ],
        out_shape=[
            jax.ShapeDtypeStruct((R, Q, BLK), jnp.float32),
            jax.ShapeDtypeStruct((NT, Q, KB // BLK), jnp.float32),
        ],
        compiler_params=pltpu.CompilerParams(
            dimension_semantics=("arbitrary",),
        ),
    )(queries, keys)

    bmax = bmax.transpose(1, 0, 2).reshape(Q, R)

    bids, grow = pl.pallas_call(
        _phase_b_body,
        out_shape=[
            jax.ShapeDtypeStruct((Q, NB), jnp.int32),
            jax.ShapeDtypeStruct((Q, NB), jnp.int32),
        ],
    )(bmax)

    cand = _sc_gather(
        sims.reshape(R * Q, BLK),
        grow.reshape(SC_NW, G_NCHUNK, G_CHUNK),
    )

    gidx = (
        bids[:, :, None] * BLK + jnp.arange(BLK, dtype=jnp.int32)
    ).reshape(Q, NB * BLK)

    QT = 256
    vals, idx, msk = pl.pallas_call(
        _phase_d_body,
        grid=(Q // QT,),
        in_specs=[
            pl.BlockSpec((QT, NB * BLK), lambda i: (i, 0)),
            pl.BlockSpec((QT, NB * BLK), lambda i: (i, 0)),
        ],
        out_specs=[
            pl.BlockSpec((QT, NB), lambda i: (i, 0)),
            pl.BlockSpec((QT, NB), lambda i: (i, 0)),
            pl.BlockSpec((QT, NB), lambda i: (i, 0)),
        ],
        out_shape=[
            jax.ShapeDtypeStruct((Q, NB), jnp.float32),
            jax.ShapeDtypeStruct((Q, NB), jnp.int32),
            jax.ShapeDtypeStruct((Q, NB), jnp.bool_),
        ],
        compiler_params=pltpu.CompilerParams(
            dimension_semantics=("arbitrary",),
        ),
    )(cand.reshape(Q, NB * BLK), gidx)

    return vals, idx, msk


# speedup vs baseline: 1.7148x; 1.1779x over previous
import functools

import jax
import jax.numpy as jnp
from jax import lax
from jax.experimental import pallas as pl
from jax.experimental.pallas import tpu as pltpu
from jax.experimental.pallas import tpu_sc as plsc

Q = 1024
D = 128
K = 100000
NB = 20
THRESH = 0.8

KB = 1024
NT = 98
KP = NT * KB
BLK = 128
R = KP // BLK
NEG = -3.0e38
BIGI = 2**30

SC_NC = 2
SC_NS = 16
SC_NW = SC_NC * SC_NS
G_TOTAL = Q * NB
G_PER_W = G_TOTAL // SC_NW
G_CHUNK = 128
G_NCHUNK = G_PER_W // G_CHUNK


def _phase_a_body(q_ref, k_ref, sims_ref, bmax_ref):
    t = pl.program_id(0)
    s = lax.dot_general(
        q_ref[...], k_ref[...],
        (((1,), (1,)), ((), ())),
        preferred_element_type=jnp.float32,
    )
    col = t * KB + lax.broadcasted_iota(jnp.int32, (Q, KB), 1)
    s = jnp.where(col < K, s, NEG)
    cols = []
    for j in range(KB // BLK):
        blk = s[:, j * BLK : (j + 1) * BLK]
        sims_ref[j, :, :] = blk
        cols.append(jnp.max(blk, axis=1, keepdims=True))
    bmax_ref[0, :, :] = jnp.concatenate(cols, axis=1).T


def _phase_b_body(m_ref, bids_ref, grow_ref):
    m = m_ref[...]
    bi = lax.broadcasted_iota(jnp.int32, (R, Q), 0)
    qi = lax.broadcasted_iota(jnp.int32, (1, Q), 1)
    sel = []
    for j in range(NB):
        mx = jnp.max(m, axis=0, keepdims=True)
        bid = jnp.min(jnp.where(m == mx, bi, BIGI), axis=0, keepdims=True)
        sel.append(bid)
        m = jnp.where(bi == bid, NEG, m)
    arr = jnp.concatenate(sel, axis=0)
    for j in range(NB):
        mn = jnp.min(arr, axis=0, keepdims=True)
        bids_ref[j : j + 1, :] = mn
        grow_ref[j : j + 1, :] = mn * Q + qi
        arr = jnp.where(arr == mn, BIGI, arr)


def _phase_d_body(cand_ref, vals_ref, pos_ref, msk_ref):
    v = cand_ref[...]
    p = lax.broadcasted_iota(jnp.int32, v.shape, 1)
    for j in range(NB):
        m = jnp.max(v, axis=1, keepdims=True)
        ci = jnp.min(jnp.where(v == m, p, BIGI), axis=1, keepdims=True)
        vals_ref[:, j : j + 1] = m
        pos_ref[:, j : j + 1] = ci
        msk_ref[:, j : j + 1] = m >= THRESH
        v = jnp.where(p == ci, NEG, v)


def _sc_gather(sims_flat, grow3d):
    mesh = plsc.VectorSubcoreMesh(core_axis_name="c", subcore_axis_name="s")

    @functools.partial(
        pl.kernel,
        mesh=mesh,
        out_type=jax.ShapeDtypeStruct((G_TOTAL, BLK), jnp.float32),
        scratch_types=[
            pltpu.VMEM((G_NCHUNK, G_CHUNK), jnp.int32),
            pltpu.VMEM((G_PER_W, BLK), jnp.float32),
            pltpu.SemaphoreType.DMA,
        ],
    )
    def gather_kernel(table_hbm, idx_hbm, out_hbm, idx_v, rows_v, sem):
        wid = lax.axis_index("s") * SC_NC + lax.axis_index("c")
        pltpu.sync_copy(idx_hbm.at[wid], idx_v)
        for j in range(G_NCHUNK):
            pltpu.async_copy(
                table_hbm.at[idx_v.at[j]],
                rows_v.at[pl.ds(j * G_CHUNK, G_CHUNK)],
                sem,
            ).wait()
        pltpu.sync_copy(rows_v, out_hbm.at[pl.ds(wid * G_PER_W, G_PER_W)])

    return gather_kernel(sims_flat, grow3d)


def kernel(queries, keys):
    sims, bmax = pl.pallas_call(
        _phase_a_body,
        grid=(NT,),
        in_specs=[
            pl.BlockSpec((Q, D), lambda t: (0, 0)),
            pl.BlockSpec((KB, D), lambda t: (t, 0)),
        ],
        out_specs=[
            pl.BlockSpec((KB // BLK, Q, BLK), lambda t: (t, 0, 0)),
            pl.BlockSpec((1, KB // BLK, Q), lambda t: (t, 0, 0)),
        ],
        out_shape=[
            jax.ShapeDtypeStruct((R, Q, BLK), jnp.float32),
            jax.ShapeDtypeStruct((NT, KB // BLK, Q), jnp.float32),
        ],
        compiler_params=pltpu.CompilerParams(
            dimension_semantics=("arbitrary",),
        ),
    )(queries, keys)

    bids, grow = pl.pallas_call(
        _phase_b_body,
        out_shape=[
            jax.ShapeDtypeStruct((NB, Q), jnp.int32),
            jax.ShapeDtypeStruct((NB, Q), jnp.int32),
        ],
    )(bmax.reshape(R, Q))

    cand = _sc_gather(
        sims.reshape(R * Q, BLK),
        grow.T.reshape(SC_NW, G_NCHUNK, G_CHUNK),
    )

    QT = 256
    vals, pos, msk = pl.pallas_call(
        _phase_d_body,
        grid=(Q // QT,),
        in_specs=[
            pl.BlockSpec((QT, NB * BLK), lambda i: (i, 0)),
        ],
        out_specs=[
            pl.BlockSpec((QT, NB), lambda i: (i, 0)),
            pl.BlockSpec((QT, NB), lambda i: (i, 0)),
            pl.BlockSpec((QT, NB), lambda i: (i, 0)),
        ],
        out_shape=[
            jax.ShapeDtypeStruct((Q, NB), jnp.float32),
            jax.ShapeDtypeStruct((Q, NB), jnp.int32),
            jax.ShapeDtypeStruct((Q, NB), jnp.bool_),
        ],
        compiler_params=pltpu.CompilerParams(
            dimension_semantics=("arbitrary",),
        ),
    )(cand.reshape(Q, NB * BLK))

    slot, lane = pos // BLK, pos % BLK
    idx = jnp.take_along_axis(bids.T, slot, axis=1) * BLK + lane
    return vals, idx, msk


# speedup vs baseline: 1.8950x; 1.1051x over previous
import functools

import jax
import jax.numpy as jnp
from jax import lax
from jax.experimental import pallas as pl
from jax.experimental.pallas import tpu as pltpu
from jax.experimental.pallas import tpu_sc as plsc

Q = 1024
D = 128
K = 100000
NB = 20
THRESH = 0.8

KB = 4096
NT = 25
KP = NT * KB
BLK = 128
R = KP // BLK
NEG = -3.0e38
BIGI = 2**30

SC_NC = 2
SC_NS = 16
SC_NW = SC_NC * SC_NS
G_TOTAL = Q * NB
G_PER_W = G_TOTAL // SC_NW
G_CHUNK = 128
G_NCHUNK = G_PER_W // G_CHUNK


def _phase_a_body(q_ref, k_ref, sims_ref, bids_ref, grow_ref, bm_ref):
    t = pl.program_id(0)
    s = lax.dot_general(
        q_ref[...], k_ref[...],
        (((1,), (1,)), ((), ())),
        preferred_element_type=jnp.float32,
    )
    col = t * KB + lax.broadcasted_iota(jnp.int32, (Q, KB), 1)
    s = jnp.where(col < K, s, NEG)
    cols = []
    for j in range(KB // BLK):
        blk = s[:, j * BLK : (j + 1) * BLK]
        sims_ref[j, :, :] = blk
        cols.append(jnp.max(blk, axis=1, keepdims=True))
    bm_ref[pl.ds(t * (KB // BLK), KB // BLK), :] = (
        jnp.concatenate(cols, axis=1).T
    )

    @pl.when(t == NT - 1)
    def _select_blocks():
        m = bm_ref[...]
        bi = lax.broadcasted_iota(jnp.int32, (R, Q), 0)
        qi = lax.broadcasted_iota(jnp.int32, (1, Q), 1)
        sel = []
        for j in range(NB):
            mx = jnp.max(m, axis=0, keepdims=True)
            bid = jnp.min(jnp.where(m == mx, bi, BIGI), axis=0, keepdims=True)
            sel.append(bid)
            m = jnp.where(bi == bid, NEG, m)
        arr = jnp.concatenate(sel, axis=0)
        for j in range(NB):
            mn = jnp.min(arr, axis=0, keepdims=True)
            bids_ref[j : j + 1, :] = mn
            grow_ref[j : j + 1, :] = mn * Q + qi
            arr = jnp.where(arr == mn, BIGI, arr)


def _phase_d_body(cand_ref, vals_ref, pos_ref, msk_ref):
    v = cand_ref[...]
    p = lax.broadcasted_iota(jnp.int32, v.shape, 1)
    for j in range(NB):
        m = jnp.max(v, axis=1, keepdims=True)
        ci = jnp.min(jnp.where(v == m, p, BIGI), axis=1, keepdims=True)
        vals_ref[:, j : j + 1] = m
        pos_ref[:, j : j + 1] = ci
        msk_ref[:, j : j + 1] = m >= THRESH
        v = jnp.where(p == ci, NEG, v)


def _sc_gather(sims_flat, grow3d):
    mesh = plsc.VectorSubcoreMesh(core_axis_name="c", subcore_axis_name="s")

    @functools.partial(
        pl.kernel,
        mesh=mesh,
        out_type=jax.ShapeDtypeStruct((G_TOTAL, BLK), jnp.float32),
        scratch_types=[
            pltpu.VMEM((G_NCHUNK, G_CHUNK), jnp.int32),
            pltpu.VMEM((G_PER_W, BLK), jnp.float32),
            pltpu.SemaphoreType.DMA,
        ],
    )
    def gather_kernel(table_hbm, idx_hbm, out_hbm, idx_v, rows_v, sem):
        wid = lax.axis_index("s") * SC_NC + lax.axis_index("c")
        pltpu.sync_copy(idx_hbm.at[wid], idx_v)
        copies = [
            pltpu.async_copy(
                table_hbm.at[idx_v.at[j]],
                rows_v.at[pl.ds(j * G_CHUNK, G_CHUNK)],
                sem,
            )
            for j in range(G_NCHUNK)
        ]
        for c in copies:
            c.wait()
        pltpu.sync_copy(rows_v, out_hbm.at[pl.ds(wid * G_PER_W, G_PER_W)])

    return gather_kernel(sims_flat, grow3d)


def kernel(queries, keys):
    sims, bids, grow = pl.pallas_call(
        _phase_a_body,
        grid=(NT,),
        in_specs=[
            pl.BlockSpec((Q, D), lambda t: (0, 0)),
            pl.BlockSpec((KB, D), lambda t: (t, 0)),
        ],
        out_specs=[
            pl.BlockSpec((KB // BLK, Q, BLK), lambda t: (t, 0, 0)),
            pl.BlockSpec((NB, Q), lambda t: (0, 0)),
            pl.BlockSpec((NB, Q), lambda t: (0, 0)),
        ],
        out_shape=[
            jax.ShapeDtypeStruct((R, Q, BLK), jnp.float32),
            jax.ShapeDtypeStruct((NB, Q), jnp.int32),
            jax.ShapeDtypeStruct((NB, Q), jnp.int32),
        ],
        scratch_shapes=[pltpu.VMEM((R, Q), jnp.float32)],
        compiler_params=pltpu.CompilerParams(
            dimension_semantics=("arbitrary",),
        ),
    )(queries, keys)

    cand = _sc_gather(
        sims.reshape(R * Q, BLK),
        grow.T.reshape(SC_NW, G_NCHUNK, G_CHUNK),
    )

    QT = 256
    vals, pos, msk = pl.pallas_call(
        _phase_d_body,
        grid=(Q // QT,),
        in_specs=[
            pl.BlockSpec((QT, NB * BLK), lambda i: (i, 0)),
        ],
        out_specs=[
            pl.BlockSpec((QT, NB), lambda i: (i, 0)),
            pl.BlockSpec((QT, NB), lambda i: (i, 0)),
            pl.BlockSpec((QT, NB), lambda i: (i, 0)),
        ],
        out_shape=[
            jax.ShapeDtypeStruct((Q, NB), jnp.float32),
            jax.ShapeDtypeStruct((Q, NB), jnp.int32),
            jax.ShapeDtypeStruct((Q, NB), jnp.bool_),
        ],
        compiler_params=pltpu.CompilerParams(
            dimension_semantics=("arbitrary",),
        ),
    )(cand.reshape(Q, NB * BLK))

    slot, lane = pos // BLK, pos % BLK
    idx = jnp.take_along_axis(bids.T, slot, axis=1) * BLK + lane
    return vals, idx, msk
